# Initial kernel scaffold; baseline (speedup 1.0000x reference)
import jax
import jax.numpy as jnp
from jax.experimental import pallas as pl


def kernel(hidden_states, w_gate, w_gate_up, w_down):
    raise NotImplementedError("write your pallas kernel here")


# speedup vs baseline: 5.0994x; 5.0994x over previous
import jax
import jax.numpy as jnp
from jax import lax
from jax.experimental import pallas as pl
from jax.experimental.pallas import tpu as pltpu

E = 64
TOPK = 2
D = 1024
F = 1024
B = 128
CHUNK = 256


def _router_body(x_ref, wg_ref, w_ref, pos_ref, be_ref):
    x = x_ref[...]
    wg = wg_ref[...]
    N = x.shape[0]
    P = TOPK * N
    NBLK = P // B + E
    logits = lax.dot_general(x, wg, (((1,), (1,)), ((), ())),
                             preferred_element_type=jnp.float32)
    scores = jax.nn.sigmoid(logits)
    iota_e = lax.broadcasted_iota(jnp.int32, (N, E), 1)
    s1 = jnp.max(scores, axis=1, keepdims=True)
    i1 = jnp.min(jnp.where(scores == s1, iota_e, E), axis=1, keepdims=True)
    masked = jnp.where(iota_e == i1, -jnp.inf, scores)
    s2 = jnp.max(masked, axis=1, keepdims=True)
    i2 = jnp.min(jnp.where(masked == s2, iota_e, E), axis=1, keepdims=True)
    sw = s1 + s2
    w_ref[...] = jnp.concatenate([s1 / sw, s2 / sw], axis=0)
    oh1 = (iota_e == i1).astype(jnp.float32)
    oh2 = (iota_e == i2).astype(jnp.float32)
    onehot = jnp.concatenate([oh1, oh2], axis=0)
    ri = lax.broadcasted_iota(jnp.int32, (CHUNK, CHUNK), 0)
    ci = lax.broadcasted_iota(jnp.int32, (CHUNK, CHUNK), 1)
    L = (ci <= ri).astype(jnp.float32)
    parts = []
    carry = jnp.zeros((1, E), jnp.float32)
    for b in range(P // CHUNK):
        seg = onehot[b * CHUNK:(b + 1) * CHUNK]
        c = jnp.dot(L, seg, preferred_element_type=jnp.float32) + carry
        parts.append(c)
        carry = c[CHUNK - 1:CHUNK, :]
    cum = jnp.concatenate(parts, axis=0)
    rank = jnp.sum(cum * onehot, axis=1, keepdims=True) - 1.0
    total = carry
    nblk = jnp.floor((total + (B - 1)) * (1.0 / B))
    re_ = lax.broadcasted_iota(jnp.int32, (E, E), 0)
    ce_ = lax.broadcasted_iota(jnp.int32, (E, E), 1)
    U = (re_ < ce_).astype(jnp.float32)
    excl = jnp.dot(nblk, U, preferred_element_type=jnp.float32)
    row_off = excl * B
    pos = jnp.sum(onehot * row_off, axis=1, keepdims=True) + rank
    pos_ref[...] = pos.astype(jnp.int32)
    ident = (re_ == ce_).astype(jnp.float32)
    excl_col = lax.dot_general(ident, excl, (((1,), (1,)), ((), ())),
                               preferred_element_type=jnp.float32)
    blk_i = lax.broadcasted_iota(jnp.int32, (E, NBLK), 1)
    le = (excl_col.astype(jnp.int32) <= blk_i).astype(jnp.float32)
    be = jnp.sum(le, axis=0, keepdims=True) - 1.0
    be_ref[...] = be.astype(jnp.int32)


def _router(x, w_gate):
    N = x.shape[0]
    P = TOPK * N
    NBLK = P // B + E
    return pl.pallas_call(
        _router_body,
        out_shape=[
            jax.ShapeDtypeStruct((P, 1), jnp.float32),
            jax.ShapeDtypeStruct((P, 1), jnp.int32),
            jax.ShapeDtypeStruct((1, NBLK), jnp.int32),
        ],
    )(x, w_gate)


def _moe_body(be_ref, xs_ref, wgu_ref, wd_ref, wp_ref, ys_ref):
    del be_ref
    gu = jnp.dot(xs_ref[...], wgu_ref[0],
                 preferred_element_type=jnp.float32)
    g = gu[:, :F]
    u = gu[:, F:]
    h = (g * jax.nn.sigmoid(g)) * u
    ys = jnp.dot(h, wd_ref[0], preferred_element_type=jnp.float32)
    ys_ref[...] = ys * wp_ref[...]


def _moe(xs_pad, w_gate_up, w_down, w_pad, be):
    NBLK = be.shape[0]
    P_PAD = xs_pad.shape[0]
    grid_spec = pltpu.PrefetchScalarGridSpec(
        num_scalar_prefetch=1,
        grid=(NBLK,),
        in_specs=[
            pl.BlockSpec((B, D), lambda i, be: (i, 0)),
            pl.BlockSpec((1, D, 2 * F), lambda i, be: (be[i], 0, 0)),
            pl.BlockSpec((1, F, D), lambda i, be: (be[i], 0, 0)),
            pl.BlockSpec((B, 1), lambda i, be: (i, 0)),
        ],
        out_specs=pl.BlockSpec((B, D), lambda i, be: (i, 0)),
    )
    return pl.pallas_call(
        _moe_body,
        grid_spec=grid_spec,
        out_shape=jax.ShapeDtypeStruct((P_PAD, D), jnp.float32),
    )(be, xs_pad, w_gate_up, w_down, w_pad)


def kernel(hidden_states, w_gate, w_gate_up, w_down):
    orig_shape = hidden_states.shape
    x = hidden_states.reshape(-1, D)
    N = x.shape[0]
    P = TOPK * N
    NBLK = P // B + E
    P_PAD = NBLK * B
    w_col, pos_col, be_row = _router(x, w_gate)
    w_flat = w_col.reshape(P)
    pos = pos_col.reshape(P)
    be = be_row.reshape(NBLK)
    x_pairs = jnp.concatenate([x, x], axis=0)
    xs_pad = jnp.zeros((P_PAD, D), jnp.float32).at[pos].set(x_pairs)
    w_pad = jnp.zeros((P_PAD, 1), jnp.float32).at[pos, 0].set(w_flat)
    ys_pad = _moe(xs_pad, w_gate_up, w_down, w_pad, be)
    ys_pairs = ys_pad[pos]
    out = ys_pairs[:N] + ys_pairs[N:]
    return out.reshape(orig_shape)
